# baseline (speedup 1.0000x reference)
import functools

import jax
import jax.numpy as jnp
from jax import lax
from jax.experimental import pallas as pl
from jax.experimental.pallas import tpu as pltpu
from jax.experimental.pallas import tpu_sc as plsc

N = 10000
E = 320000
C_IN = 128
H = 32
C_OUT = 32

NW = 32
NPW = 320
NP = NW * NPW
CH = 512
KCH = CH // 128
EP = ((E + CH + 127) // 128) * 128
RSP = NP + 16

_BL1 = 1024
_BL2 = 1024


def _mm1_body(x_ref, w_ref, b1_ref, a_ref, b_ref):
    x = x_ref[...]
    h1 = jnp.dot(x, w_ref[:C_IN, :], preferred_element_type=jnp.float32)
    h2 = jnp.dot(x, w_ref[C_IN:, :], preferred_element_type=jnp.float32)
    pad = jnp.zeros((_BL1, 128 - H), jnp.float32)
    a_ref[...] = h1
    b_ref[...] = jnp.concatenate([h2 + b1_ref[...], pad], axis=1)


_mm1 = pl.pallas_call(
    _mm1_body,
    grid=(NP // _BL1,),
    in_specs=[
        pl.BlockSpec((_BL1, C_IN), lambda i: (i, 0)),
        pl.BlockSpec((2 * C_IN, H), lambda i: (0, 0)),
        pl.BlockSpec((1, H), lambda i: (0, 0)),
    ],
    out_specs=[
        pl.BlockSpec((_BL1, H), lambda i: (i, 0)),
        pl.BlockSpec((_BL1, 128), lambda i: (i, 0)),
    ],
    out_shape=[
        jax.ShapeDtypeStruct((NP, H), jnp.float32),
        jax.ShapeDtypeStruct((NP, 128), jnp.float32),
    ],
)


def _mm2_body(s_ref, w2_ref, b2_ref, o_ref):
    s = s_ref[:, :H]
    msk = s_ref[:, H:H + 1]
    o = jnp.dot(s, w2_ref[...], preferred_element_type=jnp.float32)
    o_ref[...] = o + b2_ref[...] * msk


_mm2 = pl.pallas_call(
    _mm2_body,
    grid=(NP // _BL2,),
    in_specs=[
        pl.BlockSpec((_BL2, 128), lambda i: (i, 0)),
        pl.BlockSpec((H, C_OUT), lambda i: (0, 0)),
        pl.BlockSpec((1, C_OUT), lambda i: (0, 0)),
    ],
    out_specs=pl.BlockSpec((_BL2, C_OUT), lambda i: (i, 0)),
    out_shape=jax.ShapeDtypeStruct((N, C_OUT), jnp.float32),
)


_mesh = plsc.VectorSubcoreMesh(core_axis_name="c", subcore_axis_name="s")


@functools.partial(
    pl.kernel,
    mesh=_mesh,
    compiler_params=pltpu.CompilerParams(use_tc_tiling_on_sc=False),
    out_type=jax.ShapeDtypeStruct((NP * 128,), jnp.float32),
    scratch_types=[
        pltpu.VMEM((2 * KCH, 128), jnp.int32),
        pltpu.VMEM((2 * CH, H), jnp.float32),
        pltpu.VMEM((NPW, 128), jnp.float32),
        pltpu.VMEM((NPW + 16,), jnp.int32),
        pltpu.VMEM((NPW * 128,), jnp.float32),
        pltpu.SemaphoreType.DMA,
        pltpu.SemaphoreType.DMA,
        pltpu.SemaphoreType.DMA,
        pltpu.SemaphoreType.DMA,
    ],
)
def _sc_seg(a_hbm, bb_hbm, nbr_hbm, rs_hbm, out_hbm,
            idx_v, rows_v, bb_v, rs_v, s_v,
            rsem0, rsem1, isem0, isem1):
    nc = 2
    wid = lax.axis_index("s") * nc + lax.axis_index("c")
    n0 = wid * NPW
    pltpu.sync_copy(rs_hbm.at[pl.ds(n0, NPW + 16)], rs_v)
    pltpu.sync_copy(bb_hbm.at[pl.ds(n0, NPW)], bb_v)

    rsems = (rsem0, rsem1)
    isems = (isem0, isem1)
    zero = jnp.zeros((16,), jnp.float32)
    ones = jnp.full((16,), 1.0)

    def rs_at(i):
        return rs_v[pl.ds(i, 16)][0]

    def idx_start(b, cb):
        pltpu.async_copy(
            nbr_hbm.at[pl.ds(cb // 128, KCH)],
            idx_v.at[pl.ds(b * KCH, KCH)],
            isems[b],
        )

    def idx_wait(b):
        pltpu.make_async_copy(
            nbr_hbm.at[pl.ds(0, KCH)],
            idx_v.at[pl.ds(b * KCH, KCH)],
            isems[b],
        ).wait()

    def gathers_start(b):
        for k in range(KCH):
            pltpu.async_copy(
                a_hbm.at[idx_v.at[b * KCH + k]],
                rows_v.at[pl.ds(b * CH + k * 128, 128)],
                rsems[b],
            )

    def gathers_wait(b):
        for k in range(KCH):
            pltpu.make_async_copy(
                a_hbm.at[idx_v.at[b * KCH + k]],
                rows_v.at[pl.ds(b * CH + k * 128, 128)],
                rsems[b],
            ).wait()

    def edge_loop(off, lo, hi, bb0, bb1, j):
        dst0 = s_v.at[pl.ds(j * 128, 16)]
        dst1 = s_v.at[pl.ds(j * 128 + 16, 16)]
        nb4 = (hi - lo) // 4

        def blk(k, carry):
            p = off + lo + k * 4
            for i in range(4):
                v0 = rows_v[p + i, pl.ds(0, 16)]
                v1 = rows_v[p + i, pl.ds(16, 16)]
                plsc.addupdate(dst0, jnp.maximum(v0 + bb0, 0.0))
                plsc.addupdate(dst1, jnp.maximum(v1 + bb1, 0.0))
            return carry

        lax.fori_loop(0, nb4, blk, jnp.int32(0))

        def rem(p, carry):
            v0 = rows_v[off + p, pl.ds(0, 16)]
            v1 = rows_v[off + p, pl.ds(16, 16)]
            plsc.addupdate(dst0, jnp.maximum(v0 + bb0, 0.0))
            plsc.addupdate(dst1, jnp.maximum(v1 + bb1, 0.0))
            return carry

        lax.fori_loop(lo + nb4 * 4, hi, rem, jnp.int32(0))

    def process(off, cb, j0):
        ce = cb + CH

        def bs(_, lohi):
            lo, hi = lohi
            active = lo < hi
            mid = (lo + hi) // 2
            pred = rs_at(mid) < ce
            lo2 = jnp.where(active & pred, mid + 1, lo)
            hi2 = jnp.where(active & (~pred), mid, hi)
            return lo2, hi2

        k, _ = lax.fori_loop(0, 9, bs, (j0, jnp.int32(NPW)))

        def nbody(j, carry):
            s = rs_at(j)
            t = rs_at(j + 1)
            bb0 = bb_v[j, pl.ds(0, 16)]
            bb1 = bb_v[j, pl.ds(16, 16)]
            lo = jnp.maximum(s, cb) - cb
            hi = jnp.minimum(t, ce) - cb
            edge_loop(off, lo, hi, bb0, bb1, j)
            return carry

        lax.fori_loop(j0, k, nbody, 0)
        t_last = rs_at(k)
        return jnp.where((k > j0) & (t_last > ce), k - 1, k)

    e0 = rs_at(0)
    e1 = rs_at(NPW)
    cb0 = (e0 // 128) * 128
    nch = (e1 - cb0 + (CH - 1)) // CH

    @pl.when(nch > 0)
    def _():
        pltpu.sync_copy(nbr_hbm.at[pl.ds(cb0 // 128, KCH)], idx_v.at[pl.ds(0, KCH)])
        gathers_start(0)

    @pl.when(nch > 1)
    def _():
        idx_start(1, cb0 + CH)

    def zero_body(j, carry):
        s_v[pl.ds(j * 128, 16)] = zero
        s_v[pl.ds(j * 128 + 16, 16)] = zero
        return carry

    lax.fori_loop(0, NPW, zero_body, 0)

    def chunk_body(ci, j):
        cb = cb0 + ci * CH
        par = ci % 2
        for b in (0, 1):
            @pl.when(par == b)
            def _():
                gathers_wait(b)

            @pl.when((par == b) & (ci + 1 < nch))
            def _():
                idx_wait(1 - b)
                gathers_start(1 - b)

            @pl.when((par == b) & (ci + 2 < nch))
            def _():
                idx_start(b, cb + 2 * CH)

        return process(par * CH, cb, j)

    lax.fori_loop(0, nch, chunk_body, jnp.int32(0))

    def fin_body(j, carry):
        cnt = rs_at(j + 1) - rs_at(j)
        den = jnp.maximum(jnp.full((16,), cnt.astype(jnp.float32)), 1.0)
        s_v[pl.ds(j * 128, 16)] = s_v[pl.ds(j * 128, 16)] / den
        s_v[pl.ds(j * 128 + 16, 16)] = s_v[pl.ds(j * 128 + 16, 16)] / den
        s_v[pl.ds(j * 128 + 32, 16)] = jnp.where(cnt > 0, ones, zero)
        return carry

    lax.fori_loop(0, NPW, fin_body, 0)

    pltpu.sync_copy(s_v, out_hbm.at[pl.ds(n0 * 128, NPW * 128)])


def kernel(in_features, neighbors_index, neighbors_row_splits, W1, b1, W2, b2):
    ap, bp = _mm1(in_features, W1, b1[None, :])
    nbr2 = jnp.pad(neighbors_index, (0, EP - E)).reshape(EP // 128, 128)
    rsp = jnp.pad(
        neighbors_row_splits, (0, RSP - (N + 1)), constant_values=E
    ).astype(jnp.int32)
    s = _sc_seg(ap, bp, nbr2, rsp)
    return _mm2(s.reshape(NP, 128), W2, b2[None, :])


# speedup vs baseline: 1.5196x; 1.5196x over previous
import functools

import jax
import jax.numpy as jnp
from jax import lax
from jax.experimental import pallas as pl
from jax.experimental.pallas import tpu as pltpu
from jax.experimental.pallas import tpu_sc as plsc

N = 10000
E = 320000
C_IN = 128
H = 32
C_OUT = 32

NW = 32
NPW = 320
NP = NW * NPW
CH = 512
KCH = CH // 128
EP = ((E + CH + 127) // 128) * 128
RSP = NP + 16

_BL1 = 1024
_BL2 = 1024


def _mm1_body(x_ref, w_ref, b1_ref, a_ref, b_ref):
    x = x_ref[...]
    h1 = jnp.dot(x, w_ref[:C_IN, :], preferred_element_type=jnp.float32)
    h2 = jnp.dot(x, w_ref[C_IN:, :], preferred_element_type=jnp.float32)
    pad = jnp.zeros((_BL1, 128 - H), jnp.float32)
    a_ref[...] = h1
    b_ref[...] = jnp.concatenate([h2 + b1_ref[...], pad], axis=1)


_mm1 = pl.pallas_call(
    _mm1_body,
    grid=(NP // _BL1,),
    in_specs=[
        pl.BlockSpec((_BL1, C_IN), lambda i: (i, 0)),
        pl.BlockSpec((2 * C_IN, H), lambda i: (0, 0)),
        pl.BlockSpec((1, H), lambda i: (0, 0)),
    ],
    out_specs=[
        pl.BlockSpec((_BL1, H), lambda i: (i, 0)),
        pl.BlockSpec((_BL1, 128), lambda i: (i, 0)),
    ],
    out_shape=[
        jax.ShapeDtypeStruct((NP, H), jnp.float32),
        jax.ShapeDtypeStruct((NP, 128), jnp.float32),
    ],
)


def _mm2_body(s_ref, w2_ref, b2_ref, o_ref):
    s = s_ref[:, :H]
    msk = s_ref[:, H:H + 1]
    o = jnp.dot(s, w2_ref[...], preferred_element_type=jnp.float32)
    o_ref[...] = o + b2_ref[...] * msk


_mm2 = pl.pallas_call(
    _mm2_body,
    grid=(NP // _BL2,),
    in_specs=[
        pl.BlockSpec((_BL2, 128), lambda i: (i, 0)),
        pl.BlockSpec((H, C_OUT), lambda i: (0, 0)),
        pl.BlockSpec((1, C_OUT), lambda i: (0, 0)),
    ],
    out_specs=pl.BlockSpec((_BL2, C_OUT), lambda i: (i, 0)),
    out_shape=jax.ShapeDtypeStruct((N, C_OUT), jnp.float32),
)


_mesh = plsc.VectorSubcoreMesh(core_axis_name="c", subcore_axis_name="s")


@functools.partial(
    pl.kernel,
    mesh=_mesh,
    compiler_params=pltpu.CompilerParams(use_tc_tiling_on_sc=False),
    out_type=jax.ShapeDtypeStruct((NP * 128,), jnp.float32),
    scratch_types=[
        pltpu.VMEM((2 * KCH, 128), jnp.int32),
        pltpu.VMEM((2 * CH, H), jnp.float32),
        pltpu.VMEM((NPW, 128), jnp.float32),
        pltpu.VMEM((NPW + 16,), jnp.int32),
        pltpu.VMEM((NPW * 128,), jnp.float32),
        pltpu.SemaphoreType.DMA,
        pltpu.SemaphoreType.DMA,
        pltpu.SemaphoreType.DMA,
        pltpu.SemaphoreType.DMA,
    ],
)
def _sc_seg(a_hbm, bb_hbm, nbr_hbm, rs_hbm, out_hbm,
            idx_v, rows_v, bb_v, rs_v, s_v,
            rsem0, rsem1, isem0, isem1):
    nc = 2
    wid = lax.axis_index("s") * nc + lax.axis_index("c")
    n0 = wid * NPW
    pltpu.sync_copy(rs_hbm.at[pl.ds(n0, NPW + 16)], rs_v)
    pltpu.sync_copy(bb_hbm.at[pl.ds(n0, NPW)], bb_v)

    rsems = (rsem0, rsem1)
    isems = (isem0, isem1)
    zero = jnp.zeros((16,), jnp.float32)
    ones = jnp.full((16,), 1.0)

    def rs_at(i):
        return rs_v[pl.ds(i, 16)][0]

    def idx_start(b, cb):
        pltpu.async_copy(
            nbr_hbm.at[pl.ds(cb // 128, KCH)],
            idx_v.at[pl.ds(b * KCH, KCH)],
            isems[b],
        )

    def idx_wait(b):
        pltpu.make_async_copy(
            nbr_hbm.at[pl.ds(0, KCH)],
            idx_v.at[pl.ds(b * KCH, KCH)],
            isems[b],
        ).wait()

    def gathers_start(b):
        for k in range(KCH):
            pltpu.async_copy(
                a_hbm.at[idx_v.at[b * KCH + k]],
                rows_v.at[pl.ds(b * CH + k * 128, 128)],
                rsems[b],
            )

    def gathers_wait(b):
        for k in range(KCH):
            pltpu.make_async_copy(
                a_hbm.at[idx_v.at[b * KCH + k]],
                rows_v.at[pl.ds(b * CH + k * 128, 128)],
                rsems[b],
            ).wait()

    def edge_loop(off, lo, hi, bb0, bb1, a0, a1):
        nb4 = (hi - lo) // 4

        def blk(k, st4):
            a0, a1, c0, c1 = st4
            p = off + lo + k * 4
            v00 = rows_v[p, pl.ds(0, 16)]
            v01 = rows_v[p, pl.ds(16, 16)]
            v10 = rows_v[p + 1, pl.ds(0, 16)]
            v11 = rows_v[p + 1, pl.ds(16, 16)]
            v20 = rows_v[p + 2, pl.ds(0, 16)]
            v21 = rows_v[p + 2, pl.ds(16, 16)]
            v30 = rows_v[p + 3, pl.ds(0, 16)]
            v31 = rows_v[p + 3, pl.ds(16, 16)]
            a0 = a0 + jnp.maximum(v00 + bb0, 0.0)
            a1 = a1 + jnp.maximum(v01 + bb1, 0.0)
            c0 = c0 + jnp.maximum(v10 + bb0, 0.0)
            c1 = c1 + jnp.maximum(v11 + bb1, 0.0)
            a0 = a0 + jnp.maximum(v20 + bb0, 0.0)
            a1 = a1 + jnp.maximum(v21 + bb1, 0.0)
            c0 = c0 + jnp.maximum(v30 + bb0, 0.0)
            c1 = c1 + jnp.maximum(v31 + bb1, 0.0)
            return a0, a1, c0, c1

        a0, a1, c0, c1 = plsc.parallel_loop(
            0, nb4, 1, unroll=2, carry=(a0, a1, zero, zero)
        )(blk)

        def rem(p, st2):
            a0, a1 = st2
            v0 = rows_v[off + p, pl.ds(0, 16)]
            v1 = rows_v[off + p, pl.ds(16, 16)]
            return a0 + jnp.maximum(v0 + bb0, 0.0), a1 + jnp.maximum(v1 + bb1, 0.0)

        a0, a1 = lax.fori_loop(lo + nb4 * 4, hi, rem, (a0, a1))
        return a0 + c0, a1 + c1

    def finalize(j, cnt, a0, a1):
        den = jnp.maximum(jnp.full((16,), cnt.astype(jnp.float32)), 1.0)
        s_v[pl.ds(j * 128, 16)] = a0 / den
        s_v[pl.ds(j * 128 + 16, 16)] = a1 / den
        s_v[pl.ds(j * 128 + 32, 16)] = jnp.where(cnt > 0, ones, zero)

    def process(off, cb, j0, a00, a10):
        ce = cb + CH

        def bs(_, lohi):
            lo, hi = lohi
            active = lo < hi
            mid = (lo + hi) // 2
            pred = rs_at(mid) < ce
            lo2 = jnp.where(active & pred, mid + 1, lo)
            hi2 = jnp.where(active & (~pred), mid, hi)
            return lo2, hi2

        k, _ = lax.fori_loop(0, 9, bs, (j0, jnp.int32(NPW)))

        def nbody(j, carry):
            a0, a1 = carry
            s = rs_at(j)
            t = rs_at(j + 1)
            bb0 = bb_v[j, pl.ds(0, 16)]
            bb1 = bb_v[j, pl.ds(16, 16)]
            lo = jnp.maximum(s, cb) - cb
            hi = jnp.minimum(t, ce) - cb
            a0, a1 = edge_loop(off, lo, hi, bb0, bb1, a0, a1)
            fin = t <= ce

            @pl.when(fin)
            def _():
                finalize(j, t - s, a0, a1)

            a0 = jnp.where(fin, zero, a0)
            a1 = jnp.where(fin, zero, a1)
            return a0, a1

        a0, a1 = lax.fori_loop(j0, k, nbody, (a00, a10))
        t_last = rs_at(k)
        jn = jnp.where((k > j0) & (t_last > ce), k - 1, k)
        return jn, a0, a1

    e0 = rs_at(0)
    e1 = rs_at(NPW)
    cb0 = (e0 // 128) * 128
    nch = (e1 - cb0 + (CH - 1)) // CH

    @pl.when(nch > 0)
    def _():
        pltpu.sync_copy(nbr_hbm.at[pl.ds(cb0 // 128, KCH)], idx_v.at[pl.ds(0, KCH)])
        gathers_start(0)

    @pl.when(nch > 1)
    def _():
        idx_start(1, cb0 + CH)

    def chunk_body(ci, carry):
        j, a0, a1 = carry
        cb = cb0 + ci * CH
        par = ci % 2
        for b in (0, 1):
            @pl.when(par == b)
            def _():
                gathers_wait(b)

            @pl.when((par == b) & (ci + 1 < nch))
            def _():
                idx_wait(1 - b)
                gathers_start(1 - b)

            @pl.when((par == b) & (ci + 2 < nch))
            def _():
                idx_start(b, cb + 2 * CH)

        return process(par * CH, cb, j, a0, a1)

    jf, _, _ = lax.fori_loop(0, nch, chunk_body, (jnp.int32(0), zero, zero))

    def tail_body(j, carry):
        s_v[pl.ds(j * 128, 16)] = zero
        s_v[pl.ds(j * 128 + 16, 16)] = zero
        s_v[pl.ds(j * 128 + 32, 16)] = zero
        return carry

    lax.fori_loop(jf, NPW, tail_body, 0)

    pltpu.sync_copy(s_v, out_hbm.at[pl.ds(n0 * 128, NPW * 128)])


def kernel(in_features, neighbors_index, neighbors_row_splits, W1, b1, W2, b2):
    ap, bp = _mm1(in_features, W1, b1[None, :])
    nbr2 = jnp.pad(neighbors_index, (0, EP - E)).reshape(EP // 128, 128)
    rsp = jnp.pad(
        neighbors_row_splits, (0, RSP - (N + 1)), constant_values=E
    ).astype(jnp.int32)
    s = _sc_seg(ap, bp, nbr2, rsp)
    return _mm2(s.reshape(NP, 128), W2, b2[None, :])


# speedup vs baseline: 1.5743x; 1.0360x over previous
import functools

import jax
import jax.numpy as jnp
from jax import lax
from jax.experimental import pallas as pl
from jax.experimental.pallas import tpu as pltpu
from jax.experimental.pallas import tpu_sc as plsc

N = 10000
E = 320000
C_IN = 128
H = 32
C_OUT = 32

NW = 32
NPW = 320
NP = NW * NPW
CH = 1024
KCH = CH // 128
EP = ((E + CH + 127) // 128) * 128
RSP = NP + 16

_BL1 = 1024
_BL2 = 1024


def _mm1_body(x_ref, w_ref, b1_ref, a_ref, b_ref):
    x = x_ref[...]
    h1 = jnp.dot(x, w_ref[:C_IN, :], preferred_element_type=jnp.float32)
    h2 = jnp.dot(x, w_ref[C_IN:, :], preferred_element_type=jnp.float32)
    pad = jnp.zeros((_BL1, 128 - H), jnp.float32)
    a_ref[...] = h1
    b_ref[...] = jnp.concatenate([h2 + b1_ref[...], pad], axis=1)


_mm1 = pl.pallas_call(
    _mm1_body,
    grid=(NP // _BL1,),
    in_specs=[
        pl.BlockSpec((_BL1, C_IN), lambda i: (i, 0)),
        pl.BlockSpec((2 * C_IN, H), lambda i: (0, 0)),
        pl.BlockSpec((1, H), lambda i: (0, 0)),
    ],
    out_specs=[
        pl.BlockSpec((_BL1, H), lambda i: (i, 0)),
        pl.BlockSpec((_BL1, 128), lambda i: (i, 0)),
    ],
    out_shape=[
        jax.ShapeDtypeStruct((NP, H), jnp.float32),
        jax.ShapeDtypeStruct((NP, 128), jnp.float32),
    ],
)


def _mm2_body(s_ref, w2_ref, b2_ref, o_ref):
    s = s_ref[:, :H]
    msk = s_ref[:, H:H + 1]
    o = jnp.dot(s, w2_ref[...], preferred_element_type=jnp.float32)
    o_ref[...] = o + b2_ref[...] * msk


_mm2 = pl.pallas_call(
    _mm2_body,
    grid=(NP // _BL2,),
    in_specs=[
        pl.BlockSpec((_BL2, 128), lambda i: (i, 0)),
        pl.BlockSpec((H, C_OUT), lambda i: (0, 0)),
        pl.BlockSpec((1, C_OUT), lambda i: (0, 0)),
    ],
    out_specs=pl.BlockSpec((_BL2, C_OUT), lambda i: (i, 0)),
    out_shape=jax.ShapeDtypeStruct((N, C_OUT), jnp.float32),
)


_mesh = plsc.VectorSubcoreMesh(core_axis_name="c", subcore_axis_name="s")


@functools.partial(
    pl.kernel,
    mesh=_mesh,
    compiler_params=pltpu.CompilerParams(use_tc_tiling_on_sc=False),
    out_type=jax.ShapeDtypeStruct((NP * 128,), jnp.float32),
    scratch_types=[
        pltpu.VMEM((2 * KCH, 128), jnp.int32),
        pltpu.VMEM((2 * CH, H), jnp.float32),
        pltpu.VMEM((NPW, H), jnp.float32),
        pltpu.VMEM((NPW + 16,), jnp.int32),
        pltpu.VMEM((NPW * 128,), jnp.float32),
        pltpu.SemaphoreType.DMA,
        pltpu.SemaphoreType.DMA,
        pltpu.SemaphoreType.DMA,
        pltpu.SemaphoreType.DMA,
    ],
)
def _sc_seg(a_hbm, bb_hbm, nbr_hbm, rs_hbm, out_hbm,
            idx_v, rows_v, bb_v, rs_v, s_v,
            rsem0, rsem1, isem0, isem1):
    nc = 2
    wid = lax.axis_index("s") * nc + lax.axis_index("c")
    n0 = wid * NPW
    pltpu.sync_copy(rs_hbm.at[pl.ds(n0, NPW + 16)], rs_v)
    pltpu.sync_copy(bb_hbm.at[pl.ds(n0, NPW), pl.ds(0, H)], bb_v)

    rsems = (rsem0, rsem1)
    isems = (isem0, isem1)
    zero = jnp.zeros((16,), jnp.float32)
    ones = jnp.full((16,), 1.0)

    def rs_at(i):
        return rs_v[pl.ds(i, 16)][0]

    def idx_start(b, cb):
        pltpu.async_copy(
            nbr_hbm.at[pl.ds(cb // 128, KCH)],
            idx_v.at[pl.ds(b * KCH, KCH)],
            isems[b],
        )

    def idx_wait(b):
        pltpu.make_async_copy(
            nbr_hbm.at[pl.ds(0, KCH)],
            idx_v.at[pl.ds(b * KCH, KCH)],
            isems[b],
        ).wait()

    def gathers_start(b):
        for k in range(KCH):
            pltpu.async_copy(
                a_hbm.at[idx_v.at[b * KCH + k]],
                rows_v.at[pl.ds(b * CH + k * 128, 128)],
                rsems[b],
            )

    def gathers_wait(b):
        for k in range(KCH):
            pltpu.make_async_copy(
                a_hbm.at[idx_v.at[b * KCH + k]],
                rows_v.at[pl.ds(b * CH + k * 128, 128)],
                rsems[b],
            ).wait()

    def edge_loop(off, lo, hi, bb0, bb1, a0, a1):
        nb4 = (hi - lo) // 4

        def blk(k, st4):
            a0, a1, c0, c1 = st4
            p = off + lo + k * 4
            v00 = rows_v[p, pl.ds(0, 16)]
            v01 = rows_v[p, pl.ds(16, 16)]
            v10 = rows_v[p + 1, pl.ds(0, 16)]
            v11 = rows_v[p + 1, pl.ds(16, 16)]
            v20 = rows_v[p + 2, pl.ds(0, 16)]
            v21 = rows_v[p + 2, pl.ds(16, 16)]
            v30 = rows_v[p + 3, pl.ds(0, 16)]
            v31 = rows_v[p + 3, pl.ds(16, 16)]
            a0 = a0 + jnp.maximum(v00 + bb0, 0.0)
            a1 = a1 + jnp.maximum(v01 + bb1, 0.0)
            c0 = c0 + jnp.maximum(v10 + bb0, 0.0)
            c1 = c1 + jnp.maximum(v11 + bb1, 0.0)
            a0 = a0 + jnp.maximum(v20 + bb0, 0.0)
            a1 = a1 + jnp.maximum(v21 + bb1, 0.0)
            c0 = c0 + jnp.maximum(v30 + bb0, 0.0)
            c1 = c1 + jnp.maximum(v31 + bb1, 0.0)
            return a0, a1, c0, c1

        a0, a1, c0, c1 = plsc.parallel_loop(
            0, nb4, 1, unroll=2, carry=(a0, a1, zero, zero)
        )(blk)

        def rem(p, st2):
            a0, a1 = st2
            v0 = rows_v[off + p, pl.ds(0, 16)]
            v1 = rows_v[off + p, pl.ds(16, 16)]
            return a0 + jnp.maximum(v0 + bb0, 0.0), a1 + jnp.maximum(v1 + bb1, 0.0)

        a0, a1 = lax.fori_loop(lo + nb4 * 4, hi, rem, (a0, a1))
        return a0 + c0, a1 + c1

    def finalize(j, cnt, a0, a1):
        den = jnp.maximum(jnp.full((16,), cnt.astype(jnp.float32)), 1.0)
        s_v[pl.ds(j * 128, 16)] = a0 / den
        s_v[pl.ds(j * 128 + 16, 16)] = a1 / den
        s_v[pl.ds(j * 128 + 32, 16)] = jnp.where(cnt > 0, ones, zero)

    def process(off, cb, j0, a00, a10):
        ce = cb + CH

        def bs(_, lohi):
            lo, hi = lohi
            active = lo < hi
            mid = (lo + hi) // 2
            pred = rs_at(mid) < ce
            lo2 = jnp.where(active & pred, mid + 1, lo)
            hi2 = jnp.where(active & (~pred), mid, hi)
            return lo2, hi2

        k, _ = lax.fori_loop(0, 9, bs, (j0, jnp.int32(NPW)))

        def nbody(j, carry):
            a0, a1 = carry
            s = rs_at(j)
            t = rs_at(j + 1)
            bb0 = bb_v[j, pl.ds(0, 16)]
            bb1 = bb_v[j, pl.ds(16, 16)]
            lo = jnp.maximum(s, cb) - cb
            hi = jnp.minimum(t, ce) - cb
            a0, a1 = edge_loop(off, lo, hi, bb0, bb1, a0, a1)
            fin = t <= ce

            @pl.when(fin)
            def _():
                finalize(j, t - s, a0, a1)

            a0 = jnp.where(fin, zero, a0)
            a1 = jnp.where(fin, zero, a1)
            return a0, a1

        a0, a1 = lax.fori_loop(j0, k, nbody, (a00, a10))
        t_last = rs_at(k)
        jn = jnp.where((k > j0) & (t_last > ce), k - 1, k)
        return jn, a0, a1

    e0 = rs_at(0)
    e1 = rs_at(NPW)
    cb0 = (e0 // 128) * 128
    nch = (e1 - cb0 + (CH - 1)) // CH

    @pl.when(nch > 0)
    def _():
        pltpu.sync_copy(nbr_hbm.at[pl.ds(cb0 // 128, KCH)], idx_v.at[pl.ds(0, KCH)])
        gathers_start(0)

    @pl.when(nch > 1)
    def _():
        idx_start(1, cb0 + CH)

    def chunk_body(ci, carry):
        j, a0, a1 = carry
        cb = cb0 + ci * CH
        par = ci % 2
        for b in (0, 1):
            @pl.when(par == b)
            def _():
                gathers_wait(b)

            @pl.when((par == b) & (ci + 1 < nch))
            def _():
                idx_wait(1 - b)
                gathers_start(1 - b)

            @pl.when((par == b) & (ci + 2 < nch))
            def _():
                idx_start(b, cb + 2 * CH)

        return process(par * CH, cb, j, a0, a1)

    jf, _, _ = lax.fori_loop(0, nch, chunk_body, (jnp.int32(0), zero, zero))

    def tail_body(j, carry):
        s_v[pl.ds(j * 128, 16)] = zero
        s_v[pl.ds(j * 128 + 16, 16)] = zero
        s_v[pl.ds(j * 128 + 32, 16)] = zero
        return carry

    lax.fori_loop(jf, NPW, tail_body, 0)

    pltpu.sync_copy(s_v, out_hbm.at[pl.ds(n0 * 128, NPW * 128)])


def kernel(in_features, neighbors_index, neighbors_row_splits, W1, b1, W2, b2):
    ap, bp = _mm1(in_features, W1, b1[None, :])
    nbr2 = jnp.pad(neighbors_index, (0, EP - E)).reshape(EP // 128, 128)
    rsp = jnp.pad(
        neighbors_row_splits, (0, RSP - (N + 1)), constant_values=E
    ).astype(jnp.int32)
    s = _sc_seg(ap, bp, nbr2, rsp)
    return _mm2(s.reshape(NP, 128), W2, b2[None, :])
